# baseline (speedup 1.0000x reference)
import jax
import jax.numpy as jnp
from jax.experimental import pallas as pl
from jax.experimental.pallas import tpu as pltpu


def _fused_body(top_ref, emb_ref, h_ref, c_ref, t_ref,
                wih_ref, whh_ref, dm_ref, b_ref,
                oh_ref, oc_ref, ot_ref, ox_ref):
    f32 = jnp.float32
    x0 = emb_ref[...]
    top = top_ref[...]

    BP, S1, HL = h_ref.shape
    S = t_ref.shape[1]
    L = 2
    H = HL // L

    h_prev = h_ref[:, 0, :]
    c_prev = c_ref[:, 0, :]
    for s in range(1, S):
        sel = top == s
        h_prev = jnp.where(sel, h_ref[:, s, :], h_prev)
        c_prev = jnp.where(sel, c_ref[:, s, :], c_prev)

    wih = wih_ref[...]
    whh = whh_ref[...]
    dm = dm_ref[...]
    bias = b_ref[...]

    hi = jax.lax.Precision.HIGHEST
    x = x0
    new_h = jnp.zeros((1, 1), f32)
    new_c = jnp.zeros((1, 1), f32)
    for l in range(L):
        gates = (jnp.dot(x, wih[l], precision=hi)
                 + jnp.dot(h_prev, whh[l], precision=hi)
                 + bias[l:l + 1, :])
        i_g = jax.nn.sigmoid(gates[:, 0 * H:1 * H])
        f_g = jax.nn.sigmoid(gates[:, 1 * H:2 * H])
        g_g = jnp.tanh(gates[:, 2 * H:3 * H])
        o_g = jax.nn.sigmoid(gates[:, 3 * H:4 * H])
        c_l = jnp.dot(c_prev, dm[l], precision=hi)
        c_new = f_g * c_l + i_g * g_g
        h_new = o_g * jnp.tanh(c_new)
        h_int = jax.lax.dot_general(
            h_new, dm[l], (((1,), (1,)), ((), ())), precision=hi)
        c_int = jax.lax.dot_general(
            c_new, dm[l], (((1,), (1,)), ((), ())), precision=hi)
        if l == 0:
            new_h, new_c = h_int, c_int
        else:
            new_h, new_c = new_h + h_int, new_c + c_int
        x = h_new

    oh_ref[:, 0, :] = h_ref[:, 0, :]
    oc_ref[:, 0, :] = c_ref[:, 0, :]
    for s in range(1, S1):
        sel = top == (s - 1)
        oh_ref[:, s, :] = jnp.where(sel, new_h, h_ref[:, s, :])
        oc_ref[:, s, :] = jnp.where(sel, new_c, c_ref[:, s, :])
    for s in range(S):
        sel = top == s
        ot_ref[:, s, :] = jnp.where(sel, x0, t_ref[:, s, :])
    ox_ref[...] = x


def kernel(hiddens, cells, trees, top_position, shifted_embs,
           W_ih, W_hh, b_ih, b_hh):
    P, S1, H, L = hiddens.shape
    S, D = trees.shape[1], trees.shape[2]
    G = 4 * H
    HL = H * L
    f32 = jnp.float32

    h3 = hiddens.reshape(P, S1, HL)
    c3 = cells.reshape(P, S1, HL)
    top2 = top_position.reshape(P, 1).astype(jnp.int32)

    WhhT = jnp.transpose(W_hh, (0, 2, 1))
    WihT = jnp.transpose(W_ih, (0, 2, 1))
    eyeH = jnp.eye(H, dtype=f32)
    dm = jnp.stack([
        jnp.zeros((H, L, H), f32).at[:, l, :].set(eyeH).reshape(HL, H)
        for l in range(L)])
    whh_int = jnp.stack([
        jnp.zeros((H, L, G), f32).at[:, l, :].set(WhhT[l]).reshape(HL, G)
        for l in range(L)])
    bias = (b_ih + b_hh).astype(f32)

    BP = 32
    grid = (P // BP,)

    out_shapes = [
        jax.ShapeDtypeStruct((P, S1, HL), f32),
        jax.ShapeDtypeStruct((P, S1, HL), f32),
        jax.ShapeDtypeStruct((P, S, D), f32),
        jax.ShapeDtypeStruct((P, H), f32),
    ]

    in_specs = [
        pl.BlockSpec((BP, 1), lambda i: (i, 0)),
        pl.BlockSpec((BP, D), lambda i: (i, 0)),
        pl.BlockSpec((BP, S1, HL), lambda i: (i, 0, 0)),
        pl.BlockSpec((BP, S1, HL), lambda i: (i, 0, 0)),
        pl.BlockSpec((BP, S, D), lambda i: (i, 0, 0)),
        pl.BlockSpec((L, D, G), lambda i: (0, 0, 0)),
        pl.BlockSpec((L, HL, G), lambda i: (0, 0, 0)),
        pl.BlockSpec((L, HL, H), lambda i: (0, 0, 0)),
        pl.BlockSpec((L, G), lambda i: (0, 0)),
    ]
    out_specs = [
        pl.BlockSpec((BP, S1, HL), lambda i: (i, 0, 0)),
        pl.BlockSpec((BP, S1, HL), lambda i: (i, 0, 0)),
        pl.BlockSpec((BP, S, D), lambda i: (i, 0, 0)),
        pl.BlockSpec((BP, H), lambda i: (i, 0)),
    ]

    oh, oc, ot, ox = pl.pallas_call(
        _fused_body,
        grid=grid,
        in_specs=in_specs,
        out_specs=out_specs,
        out_shape=out_shapes,
    )(top2, shifted_embs, h3, c3, trees, WihT, whh_int, dm, bias)

    return (oh.reshape(P, S1, H, L), oc.reshape(P, S1, H, L), ot, ox)


# speedup vs baseline: 6.5963x; 6.5963x over previous
import jax
import jax.numpy as jnp
from jax.experimental import pallas as pl
from jax.experimental.pallas import tpu as pltpu

_BP = 32


def _fused_body(top_ref, emb_ref, h_ref, c_ref, t_ref,
                wih_ref, whh_ref, b_ref,
                oh_ref, oc_ref, ot_ref, ox_ref,
                hq_ref, cq_ref):
    f32 = jnp.float32
    i32 = jnp.int32
    BP = _BP
    H = emb_ref.shape[1]

    base = pl.program_id(0) * BP

    for j in range(BP):
        t = top_ref[base + j]
        src = j * 132 + t * 4
        hq_ref[pl.ds(4 * j, 4), :] = h_ref[pl.ds(src, 4), :]
        cq_ref[pl.ds(4 * j, 4), :] = c_ref[pl.ds(src, 4), :]
    hq = hq_ref[...]
    cq = cq_ref[...]

    ii = jax.lax.broadcasted_iota(i32, (BP, 4 * BP), 0)
    rr = jax.lax.broadcasted_iota(i32, (BP, 4 * BP), 1)
    sel = [(rr == 4 * ii + q).astype(f32) for q in range(4)]
    hi = jax.lax.Precision.HIGHEST
    h_prev = []
    c_prev = []
    for l in range(2):
        h_prev.append(jnp.concatenate(
            [jnp.dot(sel[l], hq, precision=hi),
             jnp.dot(sel[2 + l], hq, precision=hi)], axis=1))
        c_prev.append(jnp.concatenate(
            [jnp.dot(sel[l], cq, precision=hi),
             jnp.dot(sel[2 + l], cq, precision=hi)], axis=1))

    wih = wih_ref[...]
    whh = whh_ref[...]
    bias = b_ref[...]
    x = emb_ref[...]
    h_new = []
    c_new = []
    for l in range(2):
        gates = (jax.lax.dot_general(x, wih[l], (((1,), (1,)), ((), ())),
                                     precision=hi)
                 + jax.lax.dot_general(h_prev[l], whh[l],
                                       (((1,), (1,)), ((), ())), precision=hi)
                 + bias[l:l + 1, :])
        i_g = jax.nn.sigmoid(gates[:, 0 * H:1 * H])
        f_g = jax.nn.sigmoid(gates[:, 1 * H:2 * H])
        g_g = jnp.tanh(gates[:, 2 * H:3 * H])
        o_g = jax.nn.sigmoid(gates[:, 3 * H:4 * H])
        c_l = f_g * c_prev[l] + i_g * g_g
        h_l = o_g * jnp.tanh(c_l)
        h_new.append(h_l)
        c_new.append(c_l)
        x = h_l

    iiT = jax.lax.broadcasted_iota(i32, (4 * BP, BP), 0)
    rrT = jax.lax.broadcasted_iota(i32, (4 * BP, BP), 1)
    selT = [(iiT == 4 * rrT + q).astype(f32) for q in range(4)]
    h_pieces = [h_new[0][:, :128], h_new[1][:, :128],
                h_new[0][:, 128:], h_new[1][:, 128:]]
    c_pieces = [c_new[0][:, :128], c_new[1][:, :128],
                c_new[0][:, 128:], c_new[1][:, 128:]]
    nhq = sum(jnp.dot(selT[q], h_pieces[q], precision=hi) for q in range(4))
    ncq = sum(jnp.dot(selT[q], c_pieces[q], precision=hi) for q in range(4))

    oh_ref[...] = h_ref[...]
    oc_ref[...] = c_ref[...]
    ot_ref[...] = t_ref[...]
    emb = emb_ref[...]
    for j in range(BP):
        t = top_ref[base + j]
        dst = j * 132 + (t + 1) * 4
        oh_ref[pl.ds(dst, 4), :] = nhq[4 * j:4 * j + 4, :]
        oc_ref[pl.ds(dst, 4), :] = ncq[4 * j:4 * j + 4, :]
        r0 = j * 64 + (t >> 3) * 16 + (t & 7)
        ot_ref[pl.ds(r0, 1), :] = emb[j:j + 1, :128]
        ot_ref[pl.ds(r0 + 8, 1), :] = emb[j:j + 1, 128:]
    ox_ref[...] = x


def kernel(hiddens, cells, trees, top_position, shifted_embs,
           W_ih, W_hh, b_ih, b_hh):
    P, S1, H, L = hiddens.shape
    S, D = trees.shape[1], trees.shape[2]
    G = 4 * H
    f32 = jnp.float32
    BP = _BP

    hv = hiddens.reshape(P, S1, 2, 128, 2).transpose(0, 1, 2, 4, 3) \
                .reshape(P * S1 * 4, 128)
    cv = cells.reshape(P, S1, 2, 128, 2).transpose(0, 1, 2, 4, 3) \
              .reshape(P * S1 * 4, 128)
    tv = trees.reshape(P, 4, 8, 2, 128).transpose(0, 1, 3, 2, 4) \
              .reshape(P * 64, 128)
    top = top_position.astype(jnp.int32)
    bias = (b_ih + b_hh).astype(f32)

    grid = (P // BP,)
    out_shapes = [
        jax.ShapeDtypeStruct((P * S1 * 4, 128), f32),
        jax.ShapeDtypeStruct((P * S1 * 4, 128), f32),
        jax.ShapeDtypeStruct((P * 64, 128), f32),
        jax.ShapeDtypeStruct((P, H), f32),
    ]
    in_specs = [
        pl.BlockSpec(memory_space=pltpu.SMEM),
        pl.BlockSpec((BP, D), lambda i: (i, 0)),
        pl.BlockSpec((BP * 132, 128), lambda i: (i, 0)),
        pl.BlockSpec((BP * 132, 128), lambda i: (i, 0)),
        pl.BlockSpec((BP * 64, 128), lambda i: (i, 0)),
        pl.BlockSpec((L, G, D), lambda i: (0, 0, 0)),
        pl.BlockSpec((L, G, H), lambda i: (0, 0, 0)),
        pl.BlockSpec((L, G), lambda i: (0, 0)),
    ]
    out_specs = [
        pl.BlockSpec((BP * 132, 128), lambda i: (i, 0)),
        pl.BlockSpec((BP * 132, 128), lambda i: (i, 0)),
        pl.BlockSpec((BP * 64, 128), lambda i: (i, 0)),
        pl.BlockSpec((BP, H), lambda i: (i, 0)),
    ]

    body = _fused_body
    oh, oc, ot, ox = pl.pallas_call(
        body,
        grid=grid,
        in_specs=in_specs,
        out_specs=out_specs,
        out_shape=out_shapes,
        scratch_shapes=[pltpu.VMEM((4 * BP, 128), f32),
                        pltpu.VMEM((4 * BP, 128), f32)],
    )(top, shifted_embs, hv, cv, tv, W_ih, W_hh, bias)

    new_hiddens = oh.reshape(P, S1, 2, 2, 128).transpose(0, 1, 2, 4, 3) \
                    .reshape(P, S1, H, L)
    new_cells = oc.reshape(P, S1, 2, 2, 128).transpose(0, 1, 2, 4, 3) \
                  .reshape(P, S1, H, L)
    new_trees = ot.reshape(P, 4, 2, 8, 128).transpose(0, 1, 3, 2, 4) \
                  .reshape(P, S, D)
    return (new_hiddens, new_cells, new_trees, ox)


# speedup vs baseline: 10.1387x; 1.5370x over previous
import jax
import jax.numpy as jnp
from jax.experimental import pallas as pl
from jax.experimental.pallas import tpu as pltpu

_BP = 64


def _fused_body(top_ref, emb_ref, h_ref, c_ref, t_ref,
                wih_ref, whh_ref, b_ref,
                oh_ref, oc_ref, ot_ref, ox_ref,
                hq_ref, cq_ref, nhq_ref, ncq_ref):
    f32 = jnp.float32
    BP = _BP
    H = emb_ref.shape[1]
    base = pl.program_id(0) * BP

    for j in range(BP):
        t = top_ref[base + j]
        src = j * 132 + t * 4
        hq_ref[pl.ds(4 * j, 4), :] = h_ref[pl.ds(src, 4), :]
        cq_ref[pl.ds(4 * j, 4), :] = c_ref[pl.ds(src, 4), :]

    h_prev = []
    c_prev = []
    for l in range(2):
        h_prev.append(jnp.concatenate(
            [hq_ref[pl.Slice(l, BP, 4), :],
             hq_ref[pl.Slice(2 + l, BP, 4), :]], axis=1))
        c_prev.append(jnp.concatenate(
            [cq_ref[pl.Slice(l, BP, 4), :],
             cq_ref[pl.Slice(2 + l, BP, 4), :]], axis=1))

    wih = wih_ref[...]
    whh = whh_ref[...]
    bias = b_ref[...]
    x = emb_ref[...]
    h_new = []
    c_new = []
    for l in range(2):
        gates = (jax.lax.dot_general(x, wih[l], (((1,), (1,)), ((), ())))
                 + jax.lax.dot_general(h_prev[l], whh[l],
                                       (((1,), (1,)), ((), ())))
                 + bias[l:l + 1, :])
        i_g = jax.nn.sigmoid(gates[:, 0 * H:1 * H])
        f_g = jax.nn.sigmoid(gates[:, 1 * H:2 * H])
        g_g = jnp.tanh(gates[:, 2 * H:3 * H])
        o_g = jax.nn.sigmoid(gates[:, 3 * H:4 * H])
        c_l = f_g * c_prev[l] + i_g * g_g
        h_l = o_g * jnp.tanh(c_l)
        h_new.append(h_l)
        c_new.append(c_l)
        x = h_l

    for q, (hs, cs) in enumerate(
            [(h_new[0][:, :128], c_new[0][:, :128]),
             (h_new[1][:, :128], c_new[1][:, :128]),
             (h_new[0][:, 128:], c_new[0][:, 128:]),
             (h_new[1][:, 128:], c_new[1][:, 128:])]):
        nhq_ref[pl.Slice(q, BP, 4), :] = hs
        ncq_ref[pl.Slice(q, BP, 4), :] = cs

    oh_ref[...] = h_ref[...]
    oc_ref[...] = c_ref[...]
    ot_ref[...] = t_ref[...]
    emb = emb_ref[...]
    for j in range(BP):
        t = top_ref[base + j]
        dst = j * 132 + (t + 1) * 4
        oh_ref[pl.ds(dst, 4), :] = nhq_ref[pl.ds(4 * j, 4), :]
        oc_ref[pl.ds(dst, 4), :] = ncq_ref[pl.ds(4 * j, 4), :]
        r0 = j * 64 + (t >> 3) * 16 + (t & 7)
        ot_ref[pl.ds(r0, 1), :] = emb[j:j + 1, :128]
        ot_ref[pl.ds(r0 + 8, 1), :] = emb[j:j + 1, 128:]
    ox_ref[...] = x


def kernel(hiddens, cells, trees, top_position, shifted_embs,
           W_ih, W_hh, b_ih, b_hh):
    P, S1, H, L = hiddens.shape
    S, D = trees.shape[1], trees.shape[2]
    G = 4 * H
    f32 = jnp.float32
    BP = _BP

    hv = hiddens.reshape(P, S1, 2, 128, 2).transpose(0, 1, 2, 4, 3) \
                .reshape(P * S1 * 4, 128)
    cv = cells.reshape(P, S1, 2, 128, 2).transpose(0, 1, 2, 4, 3) \
              .reshape(P * S1 * 4, 128)
    tv = trees.reshape(P, 4, 8, 2, 128).transpose(0, 1, 3, 2, 4) \
              .reshape(P * 64, 128)
    top = top_position.astype(jnp.int32)
    bias = (b_ih + b_hh).astype(f32)

    grid = (P // BP,)
    out_shapes = [
        jax.ShapeDtypeStruct((P * S1 * 4, 128), f32),
        jax.ShapeDtypeStruct((P * S1 * 4, 128), f32),
        jax.ShapeDtypeStruct((P * 64, 128), f32),
        jax.ShapeDtypeStruct((P, H), f32),
    ]
    in_specs = [
        pl.BlockSpec(memory_space=pltpu.SMEM),
        pl.BlockSpec((BP, D), lambda i: (i, 0)),
        pl.BlockSpec((BP * 132, 128), lambda i: (i, 0)),
        pl.BlockSpec((BP * 132, 128), lambda i: (i, 0)),
        pl.BlockSpec((BP * 64, 128), lambda i: (i, 0)),
        pl.BlockSpec((L, G, D), lambda i: (0, 0, 0)),
        pl.BlockSpec((L, G, H), lambda i: (0, 0, 0)),
        pl.BlockSpec((L, G), lambda i: (0, 0)),
    ]
    out_specs = [
        pl.BlockSpec((BP * 132, 128), lambda i: (i, 0)),
        pl.BlockSpec((BP * 132, 128), lambda i: (i, 0)),
        pl.BlockSpec((BP * 64, 128), lambda i: (i, 0)),
        pl.BlockSpec((BP, H), lambda i: (i, 0)),
    ]

    oh, oc, ot, ox = pl.pallas_call(
        _fused_body,
        grid=grid,
        in_specs=in_specs,
        out_specs=out_specs,
        out_shape=out_shapes,
        scratch_shapes=[pltpu.VMEM((4 * BP, 128), f32),
                        pltpu.VMEM((4 * BP, 128), f32),
                        pltpu.VMEM((4 * BP, 128), f32),
                        pltpu.VMEM((4 * BP, 128), f32)],
    )(top, shifted_embs, hv, cv, tv, W_ih, W_hh, bias)

    new_hiddens = oh.reshape(P, S1, 2, 2, 128).transpose(0, 1, 2, 4, 3) \
                    .reshape(P, S1, H, L)
    new_cells = oc.reshape(P, S1, 2, 2, 128).transpose(0, 1, 2, 4, 3) \
                  .reshape(P, S1, H, L)
    new_trees = ot.reshape(P, 4, 2, 8, 128).transpose(0, 1, 3, 2, 4) \
                  .reshape(P, S, D)
    return (new_hiddens, new_cells, new_trees, ox)
